# Initial kernel scaffold; baseline (speedup 1.0000x reference)
import jax
import jax.numpy as jnp
from jax.experimental import pallas as pl


def kernel(triples, negs, entity_embedding, relation_embedding):
    raise NotImplementedError("write your pallas kernel here")


# speedup vs baseline: 10.5561x; 10.5561x over previous
import functools

import jax
import jax.numpy as jnp
from jax import lax
from jax.experimental import pallas as pl
from jax.experimental.pallas import tpu as pltpu
from jax.experimental.pallas import tpu_sc as plsc

B = 1024
NNEG = 256
DIM = 128
LANES = 16
NCHUNK = 128
CPR = NNEG // NCHUNK
NW = 32
BPW = B // NW
KREG = DIM // LANES


def _sc_body(heads_hbm, rels_hbm, negs_hbm, ent_hbm, rel_hbm, out_hbm,
             heads_v, rels_v, negs_v, hrow_v, rrow_v, buf0, buf1, out_v,
             sem_h, sem_r, sem_a, sem_b):
    wid = lax.axis_index("s") * 2 + lax.axis_index("c")
    base = wid * BPW

    pltpu.sync_copy(heads_hbm.at[pl.ds(base, BPW)], heads_v)
    pltpu.sync_copy(rels_hbm.at[pl.ds(base, BPW)], rels_v)
    pltpu.sync_copy(negs_hbm.at[pl.ds(base, BPW)], negs_v)

    cp_h = pltpu.async_copy(ent_hbm.at[heads_v], hrow_v, sem_h)
    cp_r = pltpu.async_copy(rel_hbm.at[rels_v], rrow_v, sem_r)
    cp_h.wait()
    cp_r.wait()

    pltpu.async_copy(ent_hbm.at[negs_v.at[0, 0]], buf0, sem_a)

    def compute_half(j, half, buf):
        h = [hrow_v[j, pl.ds(k * LANES, LANES)] *
             rrow_v[j, pl.ds(k * LANES, LANES)] for k in range(KREG)]
        lane = lax.iota(jnp.int32, LANES)

        dnums = lax.GatherDimensionNumbers(
            offset_dims=(), collapsed_slice_dims=(0,), start_index_map=(0,))

        def lanesum(v):
            for step in (8, 4, 2, 1):
                perm = lane ^ step
                v = v + lax.gather(v, perm[:, None], dnums, slice_sizes=(1,),
                                   mode=lax.GatherScatterMode.PROMISE_IN_BOUNDS)
            return v

        def grp_body(g, _):
            r = jnp.zeros((LANES,), jnp.float32)
            for i in range(LANES):
                n = g * LANES + i
                acc = buf[n, pl.ds(0, LANES)] * h[0]
                for k in range(1, KREG):
                    acc = acc + buf[n, pl.ds(k * LANES, LANES)] * h[k]
                r = jnp.where(lane == i, lanesum(acc), r)
            out_v[j, pl.ds(half * NCHUNK + g * LANES, LANES)] = r
            return 0

        lax.fori_loop(0, NCHUNK // LANES, grp_body, 0)

    def pair_body(p, _):
        pltpu.async_copy(ent_hbm.at[negs_v.at[p, 1]], buf1, sem_b)
        pltpu.make_async_copy(ent_hbm.at[negs_v.at[p, 0]], buf0, sem_a).wait()
        compute_half(p, 0, buf0)

        @pl.when(p < BPW - 1)
        def _():
            pltpu.async_copy(ent_hbm.at[negs_v.at[p + 1, 0]], buf0, sem_a)

        pltpu.make_async_copy(ent_hbm.at[negs_v.at[p, 1]], buf1, sem_b).wait()
        compute_half(p, 1, buf1)
        return 0

    lax.fori_loop(0, BPW, pair_body, 0)

    pltpu.sync_copy(out_v, out_hbm.at[pl.ds(base, BPW)])


@functools.partial(jax.jit, static_argnames=())
def _launch(heads, rels, negs3, entity_embedding, relation_embedding):
    mesh = plsc.VectorSubcoreMesh(core_axis_name="c", subcore_axis_name="s")
    return pl.kernel(
        _sc_body,
        out_type=jax.ShapeDtypeStruct((B, NNEG), jnp.float32),
        mesh=mesh,
        scratch_types=[
            pltpu.VMEM((BPW,), jnp.int32),
            pltpu.VMEM((BPW,), jnp.int32),
            pltpu.VMEM((BPW, CPR, NCHUNK), jnp.int32),
            pltpu.VMEM((BPW, DIM), jnp.float32),
            pltpu.VMEM((BPW, DIM), jnp.float32),
            pltpu.VMEM((NCHUNK, DIM), jnp.float32),
            pltpu.VMEM((NCHUNK, DIM), jnp.float32),
            pltpu.VMEM((BPW, NNEG), jnp.float32),
            pltpu.SemaphoreType.DMA,
            pltpu.SemaphoreType.DMA,
            pltpu.SemaphoreType.DMA,
            pltpu.SemaphoreType.DMA,
        ],
    )(heads, rels, negs3, entity_embedding, relation_embedding)


def kernel(triples, negs, entity_embedding, relation_embedding):
    heads = triples[:, 0].astype(jnp.int32)
    rels = triples[:, 1].astype(jnp.int32)
    negs3 = negs.astype(jnp.int32).reshape(B, CPR, NCHUNK)
    return _launch(heads, rels, negs3,
                   entity_embedding.astype(jnp.float32),
                   relation_embedding.astype(jnp.float32))


# speedup vs baseline: 12.9488x; 1.2267x over previous
import functools

import jax
import jax.numpy as jnp
from jax import lax
from jax.experimental import pallas as pl
from jax.experimental.pallas import tpu as pltpu
from jax.experimental.pallas import tpu_sc as plsc

B = 1024
NNEG = 256
DIM = 128
LANES = 16
NCHUNK = 128
CPR = NNEG // NCHUNK
NW = 32
BPW = B // NW
KREG = DIM // LANES


def _sc_body(heads_hbm, rels_hbm, negs_hbm, ent_hbm, rel_hbm, out_hbm,
             heads_v, rels_v, negs_v, hrow_v, rrow_v, buf0, buf1, buf2, buf3,
             out_v, sem_h, sem_r, sem0, sem1, sem2, sem3):
    wid = lax.axis_index("s") * 2 + lax.axis_index("c")
    base = wid * BPW

    pltpu.sync_copy(heads_hbm.at[pl.ds(base, BPW)], heads_v)
    pltpu.sync_copy(rels_hbm.at[pl.ds(base, BPW)], rels_v)
    pltpu.sync_copy(negs_hbm.at[pl.ds(base, BPW)], negs_v)

    cp_h = pltpu.async_copy(ent_hbm.at[heads_v], hrow_v, sem_h)
    cp_r = pltpu.async_copy(rel_hbm.at[rels_v], rrow_v, sem_r)
    cp_h.wait()
    cp_r.wait()

    bufs = (buf0, buf1, buf2, buf3)
    sems = (sem0, sem1, sem2, sem3)
    NBUF = 4

    for s in range(NBUF):
        pltpu.async_copy(ent_hbm.at[negs_v.at[s // CPR, s % CPR]],
                         bufs[s], sems[s])

    def compute_half(j, half, buf):
        h = [hrow_v[j, pl.ds(k * LANES, LANES)] *
             rrow_v[j, pl.ds(k * LANES, LANES)] for k in range(KREG)]
        lane = lax.iota(jnp.int32, LANES)

        dnums = lax.GatherDimensionNumbers(
            offset_dims=(), collapsed_slice_dims=(0,), start_index_map=(0,))

        def lanesum(v):
            for step in (8, 4, 2, 1):
                perm = lane ^ step
                v = v + lax.gather(v, perm[:, None], dnums, slice_sizes=(1,),
                                   mode=lax.GatherScatterMode.PROMISE_IN_BOUNDS)
            return v

        def grp_body(g, _):
            r = jnp.zeros((LANES,), jnp.float32)
            for i in range(LANES):
                n = g * LANES + i
                acc = buf[n, pl.ds(0, LANES)] * h[0]
                for k in range(1, KREG):
                    acc = acc + buf[n, pl.ds(k * LANES, LANES)] * h[k]
                r = jnp.where(lane == i, lanesum(acc), r)
            out_v[j, pl.ds(half * NCHUNK + g * LANES, LANES)] = r
            return 0

        lax.fori_loop(0, NCHUNK // LANES, grp_body, 0)

    NGRP = (BPW * CPR) // NBUF

    def grp4_body(q, _):
        for s in range(NBUF):
            j = q * (NBUF // CPR) + s // CPR
            c = s % CPR
            pltpu.make_async_copy(ent_hbm.at[negs_v.at[j, c]],
                                  bufs[s], sems[s]).wait()
            compute_half(j, c, bufs[s])

            @pl.when(q < NGRP - 1)
            def _():
                pltpu.async_copy(
                    ent_hbm.at[negs_v.at[j + (NBUF // CPR), c]],
                    bufs[s], sems[s])
        return 0

    lax.fori_loop(0, NGRP, grp4_body, 0)

    pltpu.sync_copy(out_v, out_hbm.at[pl.ds(base, BPW)])


@functools.partial(jax.jit, static_argnames=())
def _launch(heads, rels, negs3, entity_embedding, relation_embedding):
    mesh = plsc.VectorSubcoreMesh(core_axis_name="c", subcore_axis_name="s")
    return pl.kernel(
        _sc_body,
        out_type=jax.ShapeDtypeStruct((B, NNEG), jnp.float32),
        mesh=mesh,
        scratch_types=[
            pltpu.VMEM((BPW,), jnp.int32),
            pltpu.VMEM((BPW,), jnp.int32),
            pltpu.VMEM((BPW, CPR, NCHUNK), jnp.int32),
            pltpu.VMEM((BPW, DIM), jnp.float32),
            pltpu.VMEM((BPW, DIM), jnp.float32),
            pltpu.VMEM((NCHUNK, DIM), jnp.float32),
            pltpu.VMEM((NCHUNK, DIM), jnp.float32),
            pltpu.VMEM((NCHUNK, DIM), jnp.float32),
            pltpu.VMEM((NCHUNK, DIM), jnp.float32),
            pltpu.VMEM((BPW, NNEG), jnp.float32),
            pltpu.SemaphoreType.DMA,
            pltpu.SemaphoreType.DMA,
            pltpu.SemaphoreType.DMA,
            pltpu.SemaphoreType.DMA,
            pltpu.SemaphoreType.DMA,
            pltpu.SemaphoreType.DMA,
        ],
    )(heads, rels, negs3, entity_embedding, relation_embedding)


def kernel(triples, negs, entity_embedding, relation_embedding):
    heads = triples[:, 0].astype(jnp.int32)
    rels = triples[:, 1].astype(jnp.int32)
    negs3 = negs.astype(jnp.int32).reshape(B, CPR, NCHUNK)
    return _launch(heads, rels, negs3,
                   entity_embedding.astype(jnp.float32),
                   relation_embedding.astype(jnp.float32))
